# baseline (speedup 1.0000x reference)
import functools
import math

import jax
import jax.numpy as jnp
from jax import lax
from jax.experimental import pallas as pl
from jax.experimental.pallas import tpu as pltpu
from jax.experimental.pallas import tpu_sc as plsc

VOCAB = 1000000
D_MODEL = 64
SCALE = math.sqrt(D_MODEL)

NUM_CORES = 2
NUM_SUBCORES = 16
NW = NUM_CORES * NUM_SUBCORES

ROWS = 4096
SEQ = 200
BBLK = ROWS // NW
NBUF = 4


def _sc_embedding_lookup(xt, emb_weight):
    mesh = plsc.VectorSubcoreMesh(core_axis_name="c", subcore_axis_name="s")

    scratch = (
        [pltpu.VMEM((SEQ, BBLK), jnp.int32)]
        + [pltpu.VMEM((BBLK, D_MODEL), jnp.float32) for _ in range(NBUF)]
        + [pltpu.VMEM((8, 8 * BBLK), jnp.float32) for _ in range(NBUF)]
        + [pltpu.SemaphoreType.DMA for _ in range(2 * NBUF + 1)]
    )

    @functools.partial(
        pl.kernel,
        mesh=mesh,
        out_type=jax.ShapeDtypeStruct((SEQ, 8, NW, 8 * BBLK), jnp.float32),
        scratch_types=scratch,
        compiler_params=pltpu.CompilerParams(
            use_tc_tiling_on_sc=False, needs_layout_passes=False
        ),
    )
    def k(x_hbm, table_hbm, out_hbm, idx_v, *rest):
        gbuf = rest[0:NBUF]
        sbuf = rest[NBUF:2 * NBUF]
        gsem = rest[2 * NBUF:3 * NBUF]
        ssem = rest[3 * NBUF:4 * NBUF]
        isem = rest[4 * NBUF]

        wid = lax.axis_index("s") * NUM_CORES + lax.axis_index("c")

        lane = lax.iota(jnp.int32, 16)

        pltpu.make_async_copy(
            x_hbm.at[pl.ds(0, SEQ), pl.ds(wid * BBLK, BBLK)], idx_v, isem
        ).start()
        pltpu.make_async_copy(
            x_hbm.at[pl.ds(0, SEQ), pl.ds(0, BBLK)], idx_v, isem
        ).wait()

        def gather_start(u, b):
            pltpu.make_async_copy(
                table_hbm.at[idx_v.at[u]], gbuf[b], gsem[b]
            ).start()

        def gather_wait(b):
            pltpu.make_async_copy(
                table_hbm.at[idx_v.at[0]], gbuf[b], gsem[b]
            ).wait()

        def scatter_start(u, b):
            pltpu.make_async_copy(
                sbuf[b], out_hbm.at[u, pl.ds(0, 8), wid], ssem[b]
            ).start()

        def scatter_wait(b):
            pltpu.make_async_copy(
                sbuf[b], out_hbm.at[0, pl.ds(0, 8), 0], ssem[b]
            ).wait()

        for b in range(NBUF):
            gather_start(b, b)

        def transpose_scale(b):
            def diag(t, c):
                d0 = (t >> 4) << 4
                r = t & 15
                didx = d0 + ((lane + r) & 15)
                dovec = didx >> 3
                invec = ((didx & 7) << 7) + lane
                for tb in range(BBLK // 16):
                    v = plsc.load_gather(gbuf[b], [lane + (tb * 16), didx])
                    plsc.store_scatter(
                        sbuf[b], [dovec, invec + (tb * 16)], v * SCALE
                    )
                return c

            lax.fori_loop(0, (D_MODEL // 16) * 16, diag, 0, unroll=4)

        def outer(t0, carry):
            for b in range(NBUF):
                u = t0 * NBUF + b
                gather_wait(b)

                @pl.when(u >= NBUF)
                def _():
                    scatter_wait(b)

                transpose_scale(b)
                scatter_start(u, b)

                @pl.when(u + NBUF < SEQ)
                def _():
                    gather_start(u + NBUF, b)
            return carry

        lax.fori_loop(0, SEQ // NBUF, outer, 0)

        for b in range(NBUF):
            scatter_wait(b)

    return k(xt, emb_weight)


def kernel(x, emb_weight):
    out4d = _sc_embedding_lookup(x.astype(jnp.int32).T, emb_weight)
    return (
        out4d.reshape(SEQ, 8, NW, 8, BBLK)
        .transpose(2, 4, 0, 1, 3)
        .reshape(ROWS, SEQ, D_MODEL)
    )


# speedup vs baseline: 1.5277x; 1.5277x over previous
import functools
import math

import jax
import jax.numpy as jnp
from jax import lax
from jax.experimental import pallas as pl
from jax.experimental.pallas import tpu as pltpu
from jax.experimental.pallas import tpu_sc as plsc

VOCAB = 1000000
D_MODEL = 64
SCALE = math.sqrt(D_MODEL)

NUM_CORES = 2
NUM_SUBCORES = 16
NW = NUM_CORES * NUM_SUBCORES

ROWS = 4096
SEQ = 200
BBLK = ROWS // NW
NBUF = 4


def _sc_embedding_lookup(xt, emb_weight):
    mesh = plsc.VectorSubcoreMesh(core_axis_name="c", subcore_axis_name="s")

    scratch = (
        [pltpu.VMEM((SEQ, BBLK), jnp.int32)]
        + [pltpu.VMEM((BBLK, D_MODEL), jnp.float32) for _ in range(NBUF)]
        + [pltpu.VMEM((8, 8 * BBLK), jnp.float32) for _ in range(NBUF)]
        + [pltpu.SemaphoreType.DMA for _ in range(2 * NBUF + 1)]
    )

    @functools.partial(
        pl.kernel,
        mesh=mesh,
        out_type=jax.ShapeDtypeStruct((SEQ, 8, NW, 8 * BBLK), jnp.float32),
        scratch_types=scratch,
        compiler_params=pltpu.CompilerParams(
            use_tc_tiling_on_sc=False, needs_layout_passes=False
        ),
    )
    def k(x_hbm, table_hbm, out_hbm, idx_v, *rest):
        gbuf = rest[0:NBUF]
        sbuf = rest[NBUF:2 * NBUF]
        gsem = rest[2 * NBUF:3 * NBUF]
        ssem = rest[3 * NBUF:4 * NBUF]
        isem = rest[4 * NBUF]

        wid = lax.axis_index("s") * NUM_CORES + lax.axis_index("c")

        lane = lax.iota(jnp.int32, 16)

        pltpu.make_async_copy(
            x_hbm.at[pl.ds(0, SEQ), pl.ds(wid * BBLK, BBLK)], idx_v, isem
        ).start()
        pltpu.make_async_copy(
            x_hbm.at[pl.ds(0, SEQ), pl.ds(0, BBLK)], idx_v, isem
        ).wait()

        def gather_start(u, b):
            pltpu.make_async_copy(
                table_hbm.at[idx_v.at[u]], gbuf[b], gsem[b]
            ).start()

        def gather_wait(b):
            pltpu.make_async_copy(
                table_hbm.at[idx_v.at[0]], gbuf[b], gsem[b]
            ).wait()

        def scatter_start(u, b):
            pltpu.make_async_copy(
                sbuf[b], out_hbm.at[u, pl.ds(0, 8), wid], ssem[b]
            ).start()

        def scatter_wait(b):
            pltpu.make_async_copy(
                sbuf[b], out_hbm.at[0, pl.ds(0, 8), 0], ssem[b]
            ).wait()

        for b in range(NBUF):
            gather_start(b, b)

        def transpose_scale(b):
            def diag(t, c):
                d0 = (t >> 4) << 4
                r = t & 15
                didx = d0 + ((lane + r) & 15)
                dovec = didx >> 3
                invec = ((didx & 7) << 7) + lane
                vs = [
                    plsc.load_gather(gbuf[b], [lane + (tb * 16), didx])
                    for tb in range(BBLK // 16)
                ]
                for tb in range(BBLK // 16):
                    plsc.store_scatter(
                        sbuf[b], [dovec, invec + (tb * 16)], vs[tb] * SCALE
                    )
                return c

            lax.fori_loop(0, (D_MODEL // 16) * 16, diag, 0, unroll=2)

        def outer(t0, carry):
            for b in range(NBUF):
                u = t0 * NBUF + b
                gather_wait(b)

                @pl.when(u >= NBUF)
                def _():
                    scatter_wait(b)

                transpose_scale(b)
                scatter_start(u, b)

                @pl.when(u + NBUF < SEQ)
                def _():
                    gather_start(u + NBUF, b)
            return carry

        lax.fori_loop(0, SEQ // NBUF, outer, 0)

        for b in range(NBUF):
            scatter_wait(b)

    return k(xt, emb_weight)


def kernel(x, emb_weight):
    out4d = _sc_embedding_lookup(x.astype(jnp.int32).T, emb_weight)
    return (
        out4d.reshape(SEQ, 8, NW, 8, BBLK)
        .transpose(2, 4, 0, 1, 3)
        .reshape(ROWS, SEQ, D_MODEL)
    )


# speedup vs baseline: 1.5300x; 1.0015x over previous
import functools
import math

import jax
import jax.numpy as jnp
from jax import lax
from jax.experimental import pallas as pl
from jax.experimental.pallas import tpu as pltpu
from jax.experimental.pallas import tpu_sc as plsc

VOCAB = 1000000
D_MODEL = 64
SCALE = math.sqrt(D_MODEL)

NUM_CORES = 2
NUM_SUBCORES = 16
NW = NUM_CORES * NUM_SUBCORES

ROWS = 4096
SEQ = 200
BBLK = ROWS // NW
NBUF = 4


def _sc_index_descramble(xt):
    mesh = plsc.VectorSubcoreMesh(core_axis_name="c", subcore_axis_name="s")

    @functools.partial(
        pl.kernel,
        mesh=mesh,
        out_type=jax.ShapeDtypeStruct((SEQ // 8, NW, 8, BBLK), jnp.int32),
        scratch_types=[pltpu.SemaphoreType.DMA],
        compiler_params=pltpu.CompilerParams(use_tc_tiling_on_sc=True),
    )
    def k(xt_hbm, out_hbm, sem):
        wid = lax.axis_index("s") * NUM_CORES + lax.axis_index("c")
        for so in range(SEQ // 8):
            pltpu.make_async_copy(
                xt_hbm.at[pl.ds(so * 8, 8), pl.ds(wid * BBLK, BBLK)],
                out_hbm.at[so, wid],
                sem,
            ).start()
        pltpu.make_async_copy(
            xt_hbm.at[pl.ds(0, SEQ), pl.ds(0, BBLK)],
            out_hbm.at[pl.ds(0, SEQ // 8), 0],
            sem,
        ).wait()

    return k(xt)


def _sc_embedding_lookup(xin, emb_weight):
    mesh = plsc.VectorSubcoreMesh(core_axis_name="c", subcore_axis_name="s")

    scratch = (
        [pltpu.VMEM((SEQ // 8, 8, BBLK), jnp.int32)]
        + [pltpu.VMEM((BBLK, D_MODEL), jnp.float32) for _ in range(NBUF)]
        + [pltpu.VMEM((8, 8 * BBLK), jnp.float32) for _ in range(NBUF)]
        + [pltpu.SemaphoreType.DMA for _ in range(2 * NBUF + 1)]
    )

    @functools.partial(
        pl.kernel,
        mesh=mesh,
        out_type=jax.ShapeDtypeStruct((SEQ, 8, NW, 8 * BBLK), jnp.float32),
        scratch_types=scratch,
        compiler_params=pltpu.CompilerParams(
            use_tc_tiling_on_sc=False, needs_layout_passes=False
        ),
    )
    def k(xin_hbm, table_hbm, out_hbm, idx_v, *rest):
        gbuf = rest[0:NBUF]
        sbuf = rest[NBUF:2 * NBUF]
        gsem = rest[2 * NBUF:3 * NBUF]
        ssem = rest[3 * NBUF:4 * NBUF]
        isem = rest[4 * NBUF]

        wid = lax.axis_index("s") * NUM_CORES + lax.axis_index("c")

        lane = lax.iota(jnp.int32, 16)

        pltpu.make_async_copy(
            xin_hbm.at[pl.ds(0, SEQ // 8), wid], idx_v, isem
        ).start()
        pltpu.make_async_copy(
            xin_hbm.at[pl.ds(0, SEQ // 8), 0], idx_v, isem
        ).wait()

        def gather_start(u, b):
            pltpu.make_async_copy(
                table_hbm.at[idx_v.at[u >> 3, u & 7]], gbuf[b], gsem[b]
            ).start()

        def gather_wait(b):
            pltpu.make_async_copy(
                table_hbm.at[idx_v.at[0, 0]], gbuf[b], gsem[b]
            ).wait()

        def scatter_start(u, b):
            pltpu.make_async_copy(
                sbuf[b], out_hbm.at[u, pl.ds(0, 8), wid], ssem[b]
            ).start()

        def scatter_wait(b):
            pltpu.make_async_copy(
                sbuf[b], out_hbm.at[0, pl.ds(0, 8), 0], ssem[b]
            ).wait()

        for b in range(NBUF):
            gather_start(b, b)

        def transpose_scale(b):
            def diag(t, c):
                d0 = (t >> 4) << 4
                r = t & 15
                didx = d0 + ((lane + r) & 15)
                dovec = didx >> 3
                invec = ((didx & 7) << 7) + lane
                vs = [
                    plsc.load_gather(gbuf[b], [lane + (tb * 16), didx])
                    for tb in range(BBLK // 16)
                ]
                for tb in range(BBLK // 16):
                    plsc.store_scatter(
                        sbuf[b], [dovec, invec + (tb * 16)], vs[tb] * SCALE
                    )
                return c

            lax.fori_loop(0, (D_MODEL // 16) * 16, diag, 0, unroll=2)

        def outer(t0, carry):
            for b in range(NBUF):
                u = t0 * NBUF + b
                gather_wait(b)

                @pl.when(u >= NBUF)
                def _():
                    scatter_wait(b)

                transpose_scale(b)
                scatter_start(u, b)

                @pl.when(u + NBUF < SEQ)
                def _():
                    gather_start(u + NBUF, b)
            return carry

        lax.fori_loop(0, SEQ // NBUF, outer, 0)

        for b in range(NBUF):
            scatter_wait(b)

    return k(xin, emb_weight)


def kernel(x, emb_weight):
    xin = _sc_index_descramble(x.astype(jnp.int32).T)
    out4d = _sc_embedding_lookup(xin, emb_weight)
    return (
        out4d.reshape(SEQ, 8, NW, 8, BBLK)
        .transpose(2, 4, 0, 1, 3)
        .reshape(ROWS, SEQ, D_MODEL)
    )
